# Initial kernel scaffold; baseline (speedup 1.0000x reference)
import jax
import jax.numpy as jnp
from jax.experimental import pallas as pl


def kernel(input_ids, wte, wpe):
    raise NotImplementedError("write your pallas kernel here")


# speedup vs baseline: 1.4674x; 1.4674x over previous
import functools

import jax
import jax.numpy as jnp
from jax import lax
from jax.experimental import pallas as pl
from jax.experimental.pallas import tpu as pltpu
from jax.experimental.pallas import tpu_sc as plsc

_B, _S, _HID = 4, 4096, 1024
_N = _B * _S

_info = plsc.get_sparse_core_info()
_NC, _NS = _info.num_cores, _info.num_subcores
_NW = _NC * _NS
_POS_PER_W = _S // _NW
_PC = 32
_NPC = _POS_PER_W // _PC


def _sc_body(ids_hbm, wte_hbm, wpe_hbm, out_hbm, idx_v, wpe_v, rows_v, sem):
    wid = lax.axis_index("s") * _NC + lax.axis_index("c")
    p0 = wid * _POS_PER_W
    for pc in range(_NPC):
        pos = p0 + pc * _PC
        pltpu.sync_copy(wpe_hbm.at[pl.ds(pos, _PC)], wpe_v)
        for b in range(_B):
            off = b * _S + pos
            pltpu.sync_copy(ids_hbm.at[pl.ds(off, _PC)], idx_v)
            pltpu.async_copy(wte_hbm.at[idx_v], rows_v, sem).wait()

            @plsc.parallel_loop(0, _PC)
            def _add_row(i):
                for j in range(_HID // 16):
                    sl = pl.ds(j * 16, 16)
                    plsc.addupdate(rows_v.at[i, sl], wpe_v[i, sl])

            pltpu.sync_copy(rows_v, out_hbm.at[pl.ds(off, _PC)])


@jax.jit
def _embed(ids_flat, wte, wpe):
    mesh = plsc.VectorSubcoreMesh(core_axis_name="c", subcore_axis_name="s")
    return pl.kernel(
        _sc_body,
        out_type=jax.ShapeDtypeStruct((_N, _HID), jnp.float32),
        mesh=mesh,
        scratch_types=[
            pltpu.VMEM((_PC,), jnp.int32),
            pltpu.VMEM((_PC, _HID), jnp.float32),
            pltpu.VMEM((_PC, _HID), jnp.float32),
            pltpu.SemaphoreType.DMA,
        ],
    )(ids_flat, wte, wpe)


def kernel(input_ids, wte, wpe):
    ids_flat = input_ids.reshape(_N).astype(jnp.int32)
    out = _embed(ids_flat, wte, wpe)
    return out.reshape(_B, _S, _HID)


# speedup vs baseline: 2.0136x; 1.3722x over previous
import functools

import jax
import jax.numpy as jnp
from jax import lax
from jax.experimental import pallas as pl
from jax.experimental.pallas import tpu as pltpu
from jax.experimental.pallas import tpu_sc as plsc

_B, _S, _HID = 4, 4096, 1024
_N = _B * _S

_info = plsc.get_sparse_core_info()
_NC, _NS = _info.num_cores, _info.num_subcores
_NW = _NC * _NS
_POS_PER_W = _S // _NW
_PC = 32
_NPC = _POS_PER_W // _PC


_NCHUNK = _NPC * _B


def _sc_body(ids_hbm, wte_hbm, wpe_hbm, out_hbm, idx_all, wpe_v, rows0, rows1,
             sem_g, sem_o, sem_w):
    wid = lax.axis_index("s") * _NC + lax.axis_index("c")
    p0 = wid * _POS_PER_W
    bufs = (rows0, rows1)

    def idx_slice(k):
        return idx_all.at[pl.ds((k % _B) * _POS_PER_W + (k // _B) * _PC, _PC)]

    def out_slice(k):
        return out_hbm.at[pl.ds((k % _B) * _S + p0 + (k // _B) * _PC, _PC)]

    for b in range(_B):
        pltpu.sync_copy(ids_hbm.at[pl.ds(b * _S + p0, _POS_PER_W)],
                        idx_all.at[pl.ds(b * _POS_PER_W, _POS_PER_W)])

    wpe_pending = pltpu.async_copy(wpe_hbm.at[pl.ds(p0, _PC)], wpe_v, sem_w)
    gathers = [pltpu.async_copy(wte_hbm.at[idx_slice(0)], bufs[0], sem_g)]
    outs = []
    for k in range(_NCHUNK):
        buf = bufs[k % 2]
        if k % _B == 0:
            wpe_pending.wait()
        gathers[k].wait()
        if k >= 1:
            outs[k - 1].wait()
        if k + 1 < _NCHUNK:
            gathers.append(
                pltpu.async_copy(wte_hbm.at[idx_slice(k + 1)], bufs[(k + 1) % 2], sem_g))

        @plsc.parallel_loop(0, _PC)
        def _add_row(i):
            for j in range(_HID // 16):
                sl = pl.ds(j * 16, 16)
                plsc.addupdate(buf.at[i, sl], wpe_v[i, sl])

        if k % _B == _B - 1 and k + 1 < _NCHUNK:
            pos = p0 + ((k + 1) // _B) * _PC
            wpe_pending = pltpu.async_copy(wpe_hbm.at[pl.ds(pos, _PC)], wpe_v, sem_w)
        outs.append(pltpu.async_copy(buf, out_slice(k), sem_o))
    outs[-1].wait()


@jax.jit
def _embed(ids_flat, wte, wpe):
    mesh = plsc.VectorSubcoreMesh(core_axis_name="c", subcore_axis_name="s")
    return pl.kernel(
        _sc_body,
        out_type=jax.ShapeDtypeStruct((_N, _HID), jnp.float32),
        mesh=mesh,
        scratch_types=[
            pltpu.VMEM((_B * _POS_PER_W,), jnp.int32),
            pltpu.VMEM((_PC, _HID), jnp.float32),
            pltpu.VMEM((_PC, _HID), jnp.float32),
            pltpu.VMEM((_PC, _HID), jnp.float32),
            pltpu.SemaphoreType.DMA,
            pltpu.SemaphoreType.DMA,
            pltpu.SemaphoreType.DMA,
        ],
    )(ids_flat, wte, wpe)


def kernel(input_ids, wte, wpe):
    ids_flat = input_ids.reshape(_N).astype(jnp.int32)
    out = _embed(ids_flat, wte, wpe)
    return out.reshape(_B, _S, _HID)


# speedup vs baseline: 2.3443x; 1.1642x over previous
import functools

import jax
import jax.numpy as jnp
from jax import lax
from jax.experimental import pallas as pl
from jax.experimental.pallas import tpu as pltpu
from jax.experimental.pallas import tpu_sc as plsc

_B, _S, _HID = 4, 4096, 1024
_N = _B * _S

_info = plsc.get_sparse_core_info()
_NC, _NS = _info.num_cores, _info.num_subcores
_NW = _NC * _NS
_POS_PER_W = _S // _NW
_PC = 16
_NPC = _POS_PER_W // _PC
_NCHUNK = _NPC * _B
_RING = 6


def _sc_body(ids_hbm, wte_hbm, wpe_hbm, out_hbm, idx_all, wpe_v, rows,
             sem_g, sem_o, sem_w):
    wid = lax.axis_index("s") * _NC + lax.axis_index("c")
    p0 = wid * _POS_PER_W

    def idx_slice(k):
        return idx_all.at[pl.ds(lax.rem(k, _B) * _POS_PER_W
                                + (k // _B) * _PC, _PC)]

    def row_slot(k):
        return rows.at[pl.ds(lax.rem(k, _RING) * _PC, _PC)]

    def out_slice(k):
        return out_hbm.at[pl.ds(lax.rem(k, _B) * _S + p0 + (k // _B) * _PC, _PC)]

    def start_gather(k):
        pltpu.async_copy(wte_hbm.at[idx_slice(k)], row_slot(k), sem_g)

    def start_wpe(pc):
        pltpu.async_copy(wpe_hbm.at[pl.ds(p0 + pc * _PC, _PC)], wpe_v, sem_w)

    def wait(sem, dst):
        pltpu.make_async_copy(wte_hbm.at[pl.ds(0, _PC)], dst, sem).wait()

    for b in range(_B):
        pltpu.sync_copy(ids_hbm.at[pl.ds(b * _S + p0, _POS_PER_W)],
                        idx_all.at[pl.ds(b * _POS_PER_W, _POS_PER_W)])

    start_wpe(0)
    for k in range(_RING - 1):
        start_gather(k)

    def body(k, _):
        base = lax.rem(k, _RING) * _PC

        @pl.when(lax.rem(k, _B) == 0)
        def _():
            wait(sem_w, wpe_v)

        wait(sem_g, row_slot(k))

        @plsc.parallel_loop(0, _PC)
        def _add_row(i):
            for j in range(_HID // 16):
                sl = pl.ds(j * 16, 16)
                plsc.addupdate(rows.at[base + i, sl], wpe_v[i, sl])

        pltpu.async_copy(row_slot(k), out_slice(k), sem_o)

        @pl.when((lax.rem(k, _B) == _B - 1) & (k + 1 < _NCHUNK))
        def _():
            start_wpe((k + 1) // _B)

        @pl.when(k >= 1)
        def _():
            pltpu.make_async_copy(row_slot(k), out_slice(k - 1), sem_o).wait()

        @pl.when(k + _RING - 1 < _NCHUNK)
        def _():
            start_gather(k + _RING - 1)

        return ()

    lax.fori_loop(0, _NCHUNK, body, (), unroll=False)
    pltpu.make_async_copy(row_slot(0), out_slice(_NCHUNK - 1), sem_o).wait()


@jax.jit
def _embed(ids_flat, wte, wpe):
    mesh = plsc.VectorSubcoreMesh(core_axis_name="c", subcore_axis_name="s")
    return pl.kernel(
        _sc_body,
        out_type=jax.ShapeDtypeStruct((_N, _HID), jnp.float32),
        mesh=mesh,
        scratch_types=[
            pltpu.VMEM((_B * _POS_PER_W,), jnp.int32),
            pltpu.VMEM((_PC, _HID), jnp.float32),
            pltpu.VMEM((_RING * _PC, _HID), jnp.float32),
            pltpu.SemaphoreType.DMA,
            pltpu.SemaphoreType.DMA,
            pltpu.SemaphoreType.DMA,
        ],
    )(ids_flat, wte, wpe)


def kernel(input_ids, wte, wpe):
    ids_flat = input_ids.reshape(_N).astype(jnp.int32)
    out = _embed(ids_flat, wte, wpe)
    return out.reshape(_B, _S, _HID)
